# baseline (speedup 1.0000x reference)
import functools
import math

import jax
import jax.numpy as jnp
from jax import lax
from jax.experimental import pallas as pl
from jax.experimental.pallas import tpu as pltpu
from jax.experimental.pallas import tpu_sc as plsc

VOCAB_SIZE = 1000000
EMBED_DIM = 64
SCALE = math.sqrt(EMBED_DIM)

NC = 2
NS = 16
NW = NC * NS
CHUNK = 128
LANES = 16


def _make_sc_lookup(n_total: int):
    assert n_total % (NW * CHUNK) == 0
    n_chunks = n_total // (NW * CHUNK)
    mesh = plsc.VectorSubcoreMesh(core_axis_name="c", subcore_axis_name="s")

    def body(table_hbm, idx_hbm, out_hbm, idx_v, rows_v, sem):
        wid = lax.axis_index("s") * NC + lax.axis_index("c")
        pltpu.sync_copy(idx_hbm.at[wid], idx_v)

        def chunk_body(g, _):
            base = (wid * n_chunks + g) * CHUNK
            pltpu.async_copy(table_hbm.at[idx_v.at[g]], rows_v, sem).wait()

            def scale_row(r, _2):
                for j in range(EMBED_DIM // LANES):
                    sl = pl.ds(j * LANES, LANES)
                    rows_v[r, sl] = rows_v[r, sl] * SCALE
                return 0

            lax.fori_loop(0, CHUNK, scale_row, 0)
            pltpu.sync_copy(rows_v, out_hbm.at[pl.ds(base, CHUNK)])
            return 0

        lax.fori_loop(0, n_chunks, chunk_body, 0)

    return pl.kernel(
        body,
        out_type=jax.ShapeDtypeStruct((n_total, EMBED_DIM), jnp.float32),
        mesh=mesh,
        compiler_params=pltpu.CompilerParams(use_tc_tiling_on_sc=False),
        scratch_types=[
            pltpu.VMEM((n_chunks, CHUNK), jnp.int32),
            pltpu.VMEM((CHUNK, EMBED_DIM), jnp.float32),
            pltpu.SemaphoreType.DMA,
        ],
    )


@jax.jit
def kernel(x, table):
    batch, seq = x.shape
    n_total = batch * seq
    idx = x.reshape(NW, n_total // (NW * CHUNK), CHUNK).astype(jnp.int32)
    out = _make_sc_lookup(n_total)(table, idx)
    return out.reshape(batch, seq, EMBED_DIM)


# speedup vs baseline: 1.1496x; 1.1496x over previous
import functools
import math

import jax
import jax.numpy as jnp
from jax import lax
from jax.experimental import pallas as pl
from jax.experimental.pallas import tpu as pltpu
from jax.experimental.pallas import tpu_sc as plsc

VOCAB_SIZE = 1000000
EMBED_DIM = 64
SCALE = math.sqrt(EMBED_DIM)

NC = 2
NS = 16
NW = NC * NS
CHUNK = 128
NBUF = 4
LANES = 16


def _make_sc_lookup(n_total: int):
    assert n_total % (NW * CHUNK) == 0
    n_chunks = n_total // (NW * CHUNK)
    assert n_chunks % NBUF == 0
    n_super = n_chunks // NBUF
    mesh = plsc.VectorSubcoreMesh(core_axis_name="c", subcore_axis_name="s")

    def body(table_hbm, idx_hbm, out_hbm, idx_v, *bufs_and_sems):
        bufs = bufs_and_sems[:NBUF]
        g_sems = bufs_and_sems[NBUF:2 * NBUF]
        s_sems = bufs_and_sems[2 * NBUF:3 * NBUF]

        wid = lax.axis_index("s") * NC + lax.axis_index("c")
        out_base = wid * n_chunks * CHUNK
        pltpu.sync_copy(idx_hbm.at[wid], idx_v)

        def gather(g, b):
            pltpu.async_copy(table_hbm.at[idx_v.at[g]], bufs[b], g_sems[b])

        def wait_gather(g, b):
            pltpu.make_async_copy(table_hbm.at[idx_v.at[g]], bufs[b],
                                  g_sems[b]).wait()

        def scale(b):
            buf = bufs[b]

            def scale_row(r, _2):
                for j in range(EMBED_DIM // LANES):
                    sl = pl.ds(j * LANES, LANES)
                    buf[r, sl] = buf[r, sl] * SCALE
                return 0

            lax.fori_loop(0, CHUNK, scale_row, 0)

        def store(g, b):
            pltpu.async_copy(
                bufs[b], out_hbm.at[pl.ds(out_base + g * CHUNK, CHUNK)],
                s_sems[b])

        def wait_store(g, b):
            pltpu.make_async_copy(
                bufs[b], out_hbm.at[pl.ds(out_base + g * CHUNK, CHUNK)],
                s_sems[b]).wait()

        for b in range(NBUF):
            gather(b, b)

        def super_body(s, _):
            g0 = s * NBUF
            for b in range(NBUF):
                wait_gather(g0 + b, b)
                scale(b)
                store(g0 + b, b)
            for b in range(NBUF):
                wait_store(g0 + b, b)
                gather(g0 + NBUF + b, b)
            return 0

        lax.fori_loop(0, n_super - 1, super_body, 0)

        g0 = (n_super - 1) * NBUF
        for b in range(NBUF):
            wait_gather(g0 + b, b)
            scale(b)
            store(g0 + b, b)
        for b in range(NBUF):
            wait_store(g0 + b, b)

    return pl.kernel(
        body,
        out_type=jax.ShapeDtypeStruct((n_total, EMBED_DIM), jnp.float32),
        mesh=mesh,
        compiler_params=pltpu.CompilerParams(use_tc_tiling_on_sc=False),
        scratch_types=(
            [pltpu.VMEM((n_chunks, CHUNK), jnp.int32)]
            + [pltpu.VMEM((CHUNK, EMBED_DIM), jnp.float32)
               for _ in range(NBUF)]
            + [pltpu.SemaphoreType.DMA for _ in range(2 * NBUF)]
        ),
    )


@jax.jit
def kernel(x, table):
    batch, seq = x.shape
    n_total = batch * seq
    idx = x.reshape(NW, n_total // (NW * CHUNK), CHUNK).astype(jnp.int32)
    out = _make_sc_lookup(n_total)(table, idx)
    return out.reshape(batch, seq, EMBED_DIM)
